# Initial kernel scaffold; baseline (speedup 1.0000x reference)
import jax
import jax.numpy as jnp
from jax.experimental import pallas as pl


def kernel(x):
    raise NotImplementedError("write your pallas kernel here")


# speedup vs baseline: 14.3574x; 14.3574x over previous
import jax
import jax.numpy as jnp
from jax.experimental import pallas as pl
from jax.experimental.pallas import tpu as pltpu

_K = 16
_ROWS = 128


def _knn_body(xr_ref, xf_ref, dists_ref, idx_ref):
    xr = xr_ref[...]
    xf = xf_ref[...]
    n = xf.shape[0]
    inner = -2.0 * jax.lax.dot_general(
        xr.astype(jnp.bfloat16), xf.astype(jnp.bfloat16),
        (((1,), (1,)), ((), ())),
        preferred_element_type=jnp.float32,
    )
    xx_r = jnp.sum(xr * xr, axis=1, keepdims=True)
    xx_c = jnp.sum(xf * xf, axis=1)
    pd = -xx_r - inner - xx_c[None, :]

    iota = jax.lax.broadcasted_iota(jnp.int32, pd.shape, 1)
    big = jnp.int32(n)
    inf = jnp.float32(jnp.inf)
    vals = []
    inds = []
    a = pd
    for _ in range(_K + 1):
        m = jnp.min(a, axis=1, keepdims=True)
        eq = a == m
        ind = jnp.min(jnp.where(eq, iota, big), axis=1, keepdims=True)
        vals.append(m)
        inds.append(ind)
        a = jnp.where(iota == ind, inf, a)
    dists_ref[...] = jnp.concatenate(vals, axis=1)
    idx_ref[...] = jnp.concatenate(inds, axis=1)


def kernel(x):
    b, npts, d = x.shape
    n = b * npts
    xf = x.reshape(n, d)
    grid = n // _ROWS
    dists, idx = pl.pallas_call(
        _knn_body,
        grid=(grid,),
        in_specs=[
            pl.BlockSpec((_ROWS, d), lambda i: (i, 0)),
            pl.BlockSpec((n, d), lambda i: (0, 0)),
        ],
        out_specs=[
            pl.BlockSpec((_ROWS, _K + 1), lambda i: (i, 0)),
            pl.BlockSpec((_ROWS, _K + 1), lambda i: (i, 0)),
        ],
        out_shape=[
            jax.ShapeDtypeStruct((n, _K + 1), jnp.float32),
            jax.ShapeDtypeStruct((n, _K + 1), jnp.int32),
        ],
        compiler_params=pltpu.CompilerParams(
            dimension_semantics=("arbitrary",),
        ),
    )(xf, xf)
    return (
        dists[:, 1:].reshape(b, npts, _K),
        idx[:, 1:].reshape(b, npts, _K),
    )
